# baseline (speedup 1.0000x reference)
import functools

import jax
import jax.numpy as jnp
from jax import lax
from jax.experimental import pallas as pl
from jax.experimental.pallas import tpu as pltpu
from jax.experimental.pallas import tpu_sc as plsc

_NUM_FEATURES = 26
_FEATURE_SIZE = 100000
_BATCH = 4096
_EMBED_DIM = 32
_NC = 2
_NS = 16
_LANES = 16
_NW = _NC * _NS
_TOTAL = _BATCH * _NUM_FEATURES
_PER_W = _TOTAL // _NW


def _sc_body(x_hbm, table_hbm, out_hbm, idx_v, rows_v, sem):
    wid = lax.axis_index("s") * _NC + lax.axis_index("c")
    base = wid * _PER_W

    pltpu.sync_copy(x_hbm.at[pl.ds(base, _PER_W)], idx_v)

    lane = lax.iota(jnp.int32, _LANES)

    def add_offsets(c, _):
        j0 = c * _LANES
        feat = jnp.remainder(j0 + lane, _NUM_FEATURES)
        sl = pl.ds(j0, _LANES)
        idx_v[sl] = idx_v[sl] + feat * _FEATURE_SIZE
        return _

    lax.fori_loop(0, _PER_W // _LANES, add_offsets, None, unroll=8)

    pltpu.async_copy(table_hbm.at[idx_v], rows_v, sem).wait()
    pltpu.sync_copy(rows_v, out_hbm.at[pl.ds(base, _PER_W)])


@jax.jit
def kernel(x, table):
    x_flat = x.reshape(_TOTAL).astype(jnp.int32)
    mesh = plsc.VectorSubcoreMesh(
        core_axis_name="c", subcore_axis_name="s",
        num_cores=_NC, num_subcores=_NS,
    )
    out = pl.kernel(
        _sc_body,
        out_type=jax.ShapeDtypeStruct((_TOTAL, _EMBED_DIM), jnp.float32),
        mesh=mesh,
        scratch_types=[
            pltpu.VMEM((_PER_W,), jnp.int32),
            pltpu.VMEM((_PER_W, _EMBED_DIM), jnp.float32),
            pltpu.SemaphoreType.DMA,
        ],
        compiler_params=pltpu.CompilerParams(use_tc_tiling_on_sc=False),
    )(x_flat, table)
    return out.reshape(_BATCH, _NUM_FEATURES, _EMBED_DIM)
